# baseline (speedup 1.0000x reference)
import numpy as np
import jax
import jax.numpy as jnp
from jax import lax
from jax.experimental import pallas as pl
from jax.experimental.pallas import tpu as pltpu
from jax.experimental.pallas import tpu_sc as plsc

N = 320000
D = 128
G = 512
S = 5
NSLAB = N // S
NC, NS = 2, 16
NW = NC * NS
ROWS_W = NSLAB // NW
CHUNK = 80
CI = CHUNK // 2
NCH = ROWS_W // CHUNK
BM = 1600

_P2 = np.empty((D,), np.int32)
for _p in range(D):
    _j = _p % 16
    if _p < 64:
        _P2[_p] = 32 * (_p // 16) + _j
    else:
        _P2[_p] = 32 * ((_p - 64) // 16) + 16 + _j


def _gate_body(x_ref, w_ref, z_ref):
    y = jnp.dot(x_ref[...].astype(jnp.bfloat16), w_ref[...].astype(jnp.bfloat16),
                preferred_element_type=jnp.float32)
    a = y[:, :D]
    b = y[:, D:]
    zp = jnp.where(a > 0.0, a, jnp.exp(a) - 1.0) * b
    u = lax.bitcast_convert_type(zp, jnp.uint32)
    r = (u + jnp.uint32(0x7FFF) + ((u >> 16) & jnp.uint32(1))) >> 16
    w = r[:, :64] | (r[:, 64:] << 16)
    wp = jnp.concatenate([w[:BM // 2, :], w[BM // 2:, :]], axis=1)
    z_ref[...] = lax.bitcast_convert_type(wp, jnp.int32)


def _gated_matmul(x, wc, slab):
    nblk = NSLAB // BM
    return pl.pallas_call(
        _gate_body,
        grid=(nblk,),
        in_specs=[
            pl.BlockSpec((BM, D), lambda i, s=slab, n=nblk: (s * n + i, 0)),
            pl.BlockSpec((D, 2 * D), lambda i: (0, 0)),
        ],
        out_specs=pl.BlockSpec((BM // 2, D), lambda i: (i, 0)),
        out_shape=jax.ShapeDtypeStruct((NSLAB // 2, D), jnp.int32),
    )(x, wc)


def _sc_body(z_hbm, idx_hbm, zero_hbm, out_hbm,
             idx_v, zb0, zb1, zbf, stage, shared, sem0, sem1):
    c = lax.axis_index("c")
    s = lax.axis_index("s")
    wid = c * NS + s
    gs = G // NS
    pltpu.sync_copy(zero_hbm.at[pl.ds(s * gs, gs)], shared.at[pl.ds(s * gs, gs)])
    pltpu.sync_copy(idx_hbm.at[wid], idx_v)
    plsc.subcore_barrier()

    row0 = wid * (ROWS_W // 2)
    pltpu.make_async_copy(z_hbm.at[pl.ds(row0, CI)], zb0, sem0).start()
    pltpu.make_async_copy(z_hbm.at[pl.ds(row0 + CI, CI)], zb1, sem1).start()

    himask = jnp.full((16,), -65536, jnp.int32)
    sh16 = jnp.full((16,), 16, jnp.int32)

    def unpack_chunk(zb):
        def row2(k, carry):
            t0 = 2 * k
            for dt in range(2):
                t = t0 + dt
                for q in range(8):
                    v = zb[t, pl.ds(16 * q, 16)]
                    zr = t + (CI if q >= 4 else 0)
                    qq = q % 4
                    zbf[zr, pl.ds(32 * qq, 16)] = lax.bitcast_convert_type(
                        lax.shift_left(v, sh16), jnp.float32)
                    zbf[zr, pl.ds(32 * qq + 16, 16)] = lax.bitcast_convert_type(
                        jnp.bitwise_and(v, himask), jnp.float32)
            return carry
        lax.fori_loop(0, CI // 2, row2, 0)

    def process(j, zb, sem):
        pltpu.make_async_copy(z_hbm.at[pl.ds(row0 + j * CI, CI)],
                              zb, sem).wait()
        unpack_chunk(zb)

        @pl.when(j + 2 < NCH)
        def _():
            pltpu.make_async_copy(
                z_hbm.at[pl.ds(row0 + (j + 2) * CI, CI)], zb, sem
            ).start()

        pltpu.sync_copy(zbf, shared.at[idx_v.at[j]], add=True)

    def step(k, carry):
        process(2 * k, zb0, sem0)
        process(2 * k + 1, zb1, sem1)
        return carry

    lax.fori_loop(0, NCH // 2, step, 0)
    if NCH % 2:
        process(NCH - 1, zb0, sem0)
    plsc.subcore_barrier()
    pltpu.sync_copy(shared.at[pl.ds(s * gs, gs)], stage)
    pltpu.sync_copy(stage, out_hbm.at[c, pl.ds(s * gs, gs)])


def _segment_sum_sc(z, idx3, zeros):
    mesh = plsc.VectorSubcoreMesh(
        core_axis_name="c", subcore_axis_name="s",
        num_cores=NC, num_subcores=NS,
    )
    return pl.kernel(
        _sc_body,
        out_type=jax.ShapeDtypeStruct((NC, G, D), jnp.float32),
        mesh=mesh,
        scratch_types=[
            pltpu.VMEM((NCH, CHUNK), jnp.int32),
            pltpu.VMEM((CI, D), jnp.int32),
            pltpu.VMEM((CI, D), jnp.int32),
            pltpu.VMEM((CHUNK, D), jnp.float32),
            pltpu.VMEM((G // NS, D), jnp.float32),
            pltpu.VMEM_SHARED((G, D), jnp.float32),
            pltpu.SemaphoreType.DMA,
            pltpu.SemaphoreType.DMA,
        ],
    )(z, idx3, zeros)


def _merge_body(*refs):
    o_ref = refs[-1]
    acc = refs[0][0] + refs[0][1]
    for r in refs[1:-1]:
        acc = acc + r[0] + r[1]
    o_ref[...] = acc


def _merge(parts):
    return pl.pallas_call(
        _merge_body,
        out_shape=jax.ShapeDtypeStruct((G, D), jnp.float32),
    )(*parts)


def kernel(input, graph_indices, node_counts, W1, W2):
    del node_counts
    wc = jnp.concatenate([W1, W2], axis=0).T
    perm2 = jnp.asarray(np.concatenate([_P2, _P2 + D]))
    wcp = wc[:, perm2]
    nblk = NSLAB // BM
    idxr = graph_indices.astype(jnp.int32).reshape(S, nblk, 2, BM // 2)
    ia = idxr[:, :, 0, :].reshape(S, NW, NCH, CI)
    ib = idxr[:, :, 1, :].reshape(S, NW, NCH, CI)
    idx4 = jnp.stack([ia, ib], axis=3).reshape(S, NW, NCH, CHUNK)
    zeros = jnp.zeros((G, D), jnp.float32)
    parts = []
    for slab in range(S):
        z = _gated_matmul(input, wcp, slab)
        parts.append(_segment_sum_sc(z, idx4[slab], zeros))
    return _merge(parts)


# speedup vs baseline: 1.1299x; 1.1299x over previous
import jax
import jax.numpy as jnp
from jax import lax
from jax.experimental import pallas as pl
from jax.experimental.pallas import tpu as pltpu
from jax.experimental.pallas import tpu_sc as plsc

N = 320000
D = 128
G = 512
S = 5
NSLAB = N // S
NC, NS = 2, 16
NW = NC * NS
ROWS_W = NSLAB // NW
CHUNK = 80
NCH = ROWS_W // CHUNK
BM = 1600


def _gate_body(x_ref, w_ref, z_ref):
    y = jnp.dot(x_ref[...].astype(jnp.bfloat16), w_ref[...].astype(jnp.bfloat16),
                preferred_element_type=jnp.float32)
    a = y[:, :D]
    b = y[:, D:]
    z_ref[...] = jnp.where(a > 0.0, a, jnp.exp(a) - 1.0) * b


def _gated_matmul(x, wc, slab):
    nblk = NSLAB // BM
    return pl.pallas_call(
        _gate_body,
        grid=(nblk,),
        in_specs=[
            pl.BlockSpec((BM, D), lambda i, s=slab, n=nblk: (s * n + i, 0)),
            pl.BlockSpec((D, 2 * D), lambda i: (0, 0)),
        ],
        out_specs=pl.BlockSpec((BM, D), lambda i: (i, 0)),
        out_shape=jax.ShapeDtypeStruct((NSLAB, D), jnp.float32),
    )(x, wc)


def _sc_body(z_hbm, idx_hbm, zero_hbm, out_hbm,
             idx_v, zb0, zb1, zb2, zb3, stage, shared,
             sem0, sem1, sem2, sem3):
    c = lax.axis_index("c")
    s = lax.axis_index("s")
    wid = c * NS + s
    gs = G // NS
    pltpu.sync_copy(zero_hbm.at[pl.ds(s * gs, gs)], shared.at[pl.ds(s * gs, gs)])
    pltpu.sync_copy(idx_hbm.at[wid], idx_v)
    plsc.subcore_barrier()

    row0 = wid * ROWS_W
    bufs = ((zb0, sem0), (zb1, sem1), (zb2, sem2), (zb3, sem3))
    for b in range(4):
        pltpu.make_async_copy(z_hbm.at[pl.ds(row0 + b * CHUNK, CHUNK)],
                              bufs[b][0], bufs[b][1]).start()

    def one(j, zb, sem):
        pltpu.make_async_copy(z_hbm.at[pl.ds(row0 + j * CHUNK, CHUNK)],
                              zb, sem).wait()
        pltpu.sync_copy(zb, shared.at[idx_v.at[j]], add=True)

        @pl.when(j + 4 < NCH)
        def _():
            pltpu.make_async_copy(
                z_hbm.at[pl.ds(row0 + (j + 4) * CHUNK, CHUNK)], zb, sem
            ).start()

    def step(k, carry):
        j0 = 4 * k
        for b in range(4):
            one(j0 + b, bufs[b][0], bufs[b][1])
        return carry

    lax.fori_loop(0, NCH // 4, step, 0)
    for r in range(NCH % 4):
        jt = (NCH // 4) * 4 + r
        one(jt, bufs[r][0], bufs[r][1])
    plsc.subcore_barrier()
    pltpu.sync_copy(shared.at[pl.ds(s * gs, gs)], stage)
    pltpu.sync_copy(stage, out_hbm.at[c, pl.ds(s * gs, gs)])


def _segment_sum_sc(z, idx3, zeros):
    mesh = plsc.VectorSubcoreMesh(
        core_axis_name="c", subcore_axis_name="s",
        num_cores=NC, num_subcores=NS,
    )
    return pl.kernel(
        _sc_body,
        out_type=jax.ShapeDtypeStruct((NC, G, D), jnp.float32),
        mesh=mesh,
        scratch_types=[
            pltpu.VMEM((NCH, CHUNK), jnp.int32),
            pltpu.VMEM((CHUNK, D), jnp.float32),
            pltpu.VMEM((CHUNK, D), jnp.float32),
            pltpu.VMEM((CHUNK, D), jnp.float32),
            pltpu.VMEM((CHUNK, D), jnp.float32),
            pltpu.VMEM((G // NS, D), jnp.float32),
            pltpu.VMEM_SHARED((G, D), jnp.float32),
            pltpu.SemaphoreType.DMA,
            pltpu.SemaphoreType.DMA,
            pltpu.SemaphoreType.DMA,
            pltpu.SemaphoreType.DMA,
        ],
    )(z, idx3, zeros)


def _merge_body(*refs):
    o_ref = refs[-1]
    acc = refs[0][0] + refs[0][1]
    for r in refs[1:-1]:
        acc = acc + r[0] + r[1]
    o_ref[...] = acc


def _merge(parts):
    return pl.pallas_call(
        _merge_body,
        out_shape=jax.ShapeDtypeStruct((G, D), jnp.float32),
    )(*parts)


def kernel(input, graph_indices, node_counts, W1, W2):
    del node_counts
    wc = jnp.concatenate([W1, W2], axis=0).T
    idx4 = graph_indices.astype(jnp.int32).reshape(S, NW, NCH, CHUNK)
    zeros = jnp.zeros((G, D), jnp.float32)
    parts = []
    for slab in range(S):
        z = _gated_matmul(input, wc, slab)
        parts.append(_segment_sum_sc(z, idx4[slab], zeros))
    return _merge(parts)


# speedup vs baseline: 1.1346x; 1.0041x over previous
import jax
import jax.numpy as jnp
from jax import lax
from jax.experimental import pallas as pl
from jax.experimental.pallas import tpu as pltpu
from jax.experimental.pallas import tpu_sc as plsc

N = 320000
D = 128
G = 512
S = 5
NSLAB = N // S
NC, NS = 2, 16
NW = NC * NS
ROWS_W = NSLAB // NW
CHUNK = 80
NCH = ROWS_W // CHUNK
BM = 1600


def _gate_body(x_ref, w_ref, z_ref):
    y = jnp.dot(x_ref[...].astype(jnp.bfloat16), w_ref[...].astype(jnp.bfloat16),
                preferred_element_type=jnp.float32)
    a = y[:, :D]
    b = y[:, D:]
    z_ref[...] = jnp.where(a > 0.0, a, jnp.exp(a) - 1.0) * b


def _gated_matmul(x, wc, slab):
    nblk = NSLAB // BM
    return pl.pallas_call(
        _gate_body,
        grid=(nblk,),
        in_specs=[
            pl.BlockSpec((BM, D), lambda i, s=slab, n=nblk: (s * n + i, 0)),
            pl.BlockSpec((D, 2 * D), lambda i: (0, 0)),
        ],
        out_specs=pl.BlockSpec((BM, D), lambda i: (i, 0)),
        out_shape=jax.ShapeDtypeStruct((NSLAB, D), jnp.float32),
    )(x, wc)


def _sc_body(z_hbm, idx_hbm, zero_hbm, out_hbm,
             idx_v, zb0, zb1, stage, shared, sem0, sem1):
    c = lax.axis_index("c")
    s = lax.axis_index("s")
    wid = c * NS + s
    gs = G // NS
    pltpu.sync_copy(zero_hbm.at[pl.ds(s * gs, gs)], shared.at[pl.ds(s * gs, gs)])
    pltpu.sync_copy(zero_hbm.at[pl.ds(s * gs, gs)],
                    shared.at[pl.ds(G + s * gs, gs)])
    pltpu.sync_copy(idx_hbm.at[wid], idx_v)
    off512 = jnp.full((16,), G, jnp.int32)

    @pl.when(s % 2 == 1)
    def _():
        def addoff(i, carry):
            for m in range(CHUNK // 16):
                idx_v[i, pl.ds(16 * m, 16)] = (
                    idx_v[i, pl.ds(16 * m, 16)] + off512)
            return carry
        lax.fori_loop(0, NCH, addoff, 0)

    plsc.subcore_barrier()

    row0 = wid * ROWS_W
    pltpu.make_async_copy(z_hbm.at[pl.ds(row0, CHUNK)], zb0, sem0).start()
    pltpu.make_async_copy(z_hbm.at[pl.ds(row0 + CHUNK, CHUNK)], zb1, sem1).start()

    def step(k, carry):
        j0 = 2 * k
        pltpu.make_async_copy(z_hbm.at[pl.ds(row0 + j0 * CHUNK, CHUNK)],
                              zb0, sem0).wait()
        pltpu.sync_copy(zb0, shared.at[idx_v.at[j0]], add=True)

        @pl.when(j0 + 2 < NCH)
        def _():
            pltpu.make_async_copy(
                z_hbm.at[pl.ds(row0 + (j0 + 2) * CHUNK, CHUNK)], zb0, sem0
            ).start()

        pltpu.make_async_copy(z_hbm.at[pl.ds(row0 + (j0 + 1) * CHUNK, CHUNK)],
                              zb1, sem1).wait()
        pltpu.sync_copy(zb1, shared.at[idx_v.at[j0 + 1]], add=True)

        @pl.when(j0 + 3 < NCH)
        def _():
            pltpu.make_async_copy(
                z_hbm.at[pl.ds(row0 + (j0 + 3) * CHUNK, CHUNK)], zb1, sem1
            ).start()

        return carry

    lax.fori_loop(0, NCH // 2, step, 0)
    if NCH % 2:
        jt = NCH - 1
        pltpu.make_async_copy(z_hbm.at[pl.ds(row0 + jt * CHUNK, CHUNK)],
                              zb0, sem0).wait()
        pltpu.sync_copy(zb0, shared.at[idx_v.at[jt]], add=True)
    plsc.subcore_barrier()
    pltpu.sync_copy(shared.at[pl.ds(s * gs, gs)], stage)
    pltpu.sync_copy(stage, out_hbm.at[c, 0, pl.ds(s * gs, gs)])
    pltpu.sync_copy(shared.at[pl.ds(G + s * gs, gs)], stage)
    pltpu.sync_copy(stage, out_hbm.at[c, 1, pl.ds(s * gs, gs)])


def _segment_sum_sc(z, idx3, zeros):
    mesh = plsc.VectorSubcoreMesh(
        core_axis_name="c", subcore_axis_name="s",
        num_cores=NC, num_subcores=NS,
    )
    return pl.kernel(
        _sc_body,
        out_type=jax.ShapeDtypeStruct((NC, 2, G, D), jnp.float32),
        mesh=mesh,
        scratch_types=[
            pltpu.VMEM((NCH, CHUNK), jnp.int32),
            pltpu.VMEM((CHUNK, D), jnp.float32),
            pltpu.VMEM((CHUNK, D), jnp.float32),
            pltpu.VMEM((G // NS, D), jnp.float32),
            pltpu.VMEM_SHARED((2 * G, D), jnp.float32),
            pltpu.SemaphoreType.DMA,
            pltpu.SemaphoreType.DMA,
        ],
    )(z, idx3, zeros)


def _merge_body(*refs):
    o_ref = refs[-1]
    acc = None
    for r in refs[:-1]:
        for part in (r[0, 0], r[0, 1], r[1, 0], r[1, 1]):
            acc = part if acc is None else acc + part
    o_ref[...] = acc


def _merge(parts):
    return pl.pallas_call(
        _merge_body,
        out_shape=jax.ShapeDtypeStruct((G, D), jnp.float32),
    )(*parts)


def kernel(input, graph_indices, node_counts, W1, W2):
    del node_counts
    wc = jnp.concatenate([W1, W2], axis=0).T
    idx4 = graph_indices.astype(jnp.int32).reshape(S, NW, NCH, CHUNK)
    zeros = jnp.zeros((G, D), jnp.float32)
    parts = []
    for slab in range(S):
        z = _gated_matmul(input, wc, slab)
        parts.append(_segment_sum_sc(z, idx4[slab], zeros))
    return _merge(parts)


# speedup vs baseline: 1.1430x; 1.0074x over previous
import jax
import jax.numpy as jnp
from jax import lax
from jax.experimental import pallas as pl
from jax.experimental.pallas import tpu as pltpu
from jax.experimental.pallas import tpu_sc as plsc

N = 320000
D = 128
G = 512
S = 5
NSLAB = N // S
NC, NS = 2, 16
NW = NC * NS
ROWS_W = NSLAB // NW
CHUNK = 80
NCH = ROWS_W // CHUNK
BM = 1600


def _gate_body(x_ref, w_ref, z_ref):
    y = jnp.dot(x_ref[...].astype(jnp.bfloat16), w_ref[...].astype(jnp.bfloat16),
                preferred_element_type=jnp.float32)
    a = y[:, :D]
    b = y[:, D:]
    z_ref[...] = jnp.where(a > 0.0, a, jnp.exp(a) - 1.0) * b


def _gated_matmul(x, wc, slab):
    nblk = NSLAB // BM
    return pl.pallas_call(
        _gate_body,
        grid=(nblk,),
        in_specs=[
            pl.BlockSpec((BM, D), lambda i, s=slab, n=nblk: (s * n + i, 0)),
            pl.BlockSpec((D, 2 * D), lambda i: (0, 0)),
        ],
        out_specs=pl.BlockSpec((BM, D), lambda i: (i, 0)),
        out_shape=jax.ShapeDtypeStruct((NSLAB, D), jnp.float32),
    )(x, wc)


def _sc_body(z_hbm, idx_hbm, zero_hbm, out_hbm,
             idx_v, zb0, zb1, stage, shared, sem0, sem1):
    c = lax.axis_index("c")
    s = lax.axis_index("s")
    wid = c * NS + s
    gs = G // NS
    pltpu.sync_copy(zero_hbm.at[pl.ds(s * gs, gs)], shared.at[pl.ds(s * gs, gs)])
    pltpu.sync_copy(idx_hbm.at[wid], idx_v)
    plsc.subcore_barrier()

    row0 = wid * ROWS_W
    pltpu.make_async_copy(z_hbm.at[pl.ds(row0, CHUNK)], zb0, sem0).start()
    pltpu.make_async_copy(z_hbm.at[pl.ds(row0 + CHUNK, CHUNK)], zb1, sem1).start()

    def step(k, carry):
        j0 = 2 * k
        pltpu.make_async_copy(z_hbm.at[pl.ds(row0 + j0 * CHUNK, CHUNK)],
                              zb0, sem0).wait()
        pltpu.sync_copy(zb0, shared.at[idx_v.at[j0]], add=True)

        @pl.when(j0 + 2 < NCH)
        def _():
            pltpu.make_async_copy(
                z_hbm.at[pl.ds(row0 + (j0 + 2) * CHUNK, CHUNK)], zb0, sem0
            ).start()

        pltpu.make_async_copy(z_hbm.at[pl.ds(row0 + (j0 + 1) * CHUNK, CHUNK)],
                              zb1, sem1).wait()
        pltpu.sync_copy(zb1, shared.at[idx_v.at[j0 + 1]], add=True)

        @pl.when(j0 + 3 < NCH)
        def _():
            pltpu.make_async_copy(
                z_hbm.at[pl.ds(row0 + (j0 + 3) * CHUNK, CHUNK)], zb1, sem1
            ).start()

        return carry

    lax.fori_loop(0, NCH // 2, step, 0)
    if NCH % 2:
        jt = NCH - 1
        pltpu.make_async_copy(z_hbm.at[pl.ds(row0 + jt * CHUNK, CHUNK)],
                              zb0, sem0).wait()
        pltpu.sync_copy(zb0, shared.at[idx_v.at[jt]], add=True)
    plsc.subcore_barrier()
    pltpu.sync_copy(shared.at[pl.ds(s * gs, gs)], stage)
    pltpu.sync_copy(stage, out_hbm.at[c, pl.ds(s * gs, gs)])


def _segment_sum_sc(z, idx3, zeros):
    mesh = plsc.VectorSubcoreMesh(
        core_axis_name="c", subcore_axis_name="s",
        num_cores=NC, num_subcores=NS,
    )
    return pl.kernel(
        _sc_body,
        out_type=jax.ShapeDtypeStruct((NC, G, D), jnp.float32),
        mesh=mesh,
        scratch_types=[
            pltpu.VMEM((NCH, CHUNK), jnp.int32),
            pltpu.VMEM((CHUNK, D), jnp.float32),
            pltpu.VMEM((CHUNK, D), jnp.float32),
            pltpu.VMEM((G // NS, D), jnp.float32),
            pltpu.VMEM_SHARED((G, D), jnp.float32),
            pltpu.SemaphoreType.DMA,
            pltpu.SemaphoreType.DMA,
        ],
    )(z, idx3, zeros)


def _merge_body(*refs):
    o_ref = refs[-1]
    acc = refs[0][0] + refs[0][1]
    for r in refs[1:-1]:
        acc = acc + r[0] + r[1]
    o_ref[...] = acc


def _merge(parts):
    return pl.pallas_call(
        _merge_body,
        out_shape=jax.ShapeDtypeStruct((G, D), jnp.float32),
    )(*parts)


def kernel(input, graph_indices, node_counts, W1, W2):
    del node_counts
    wc = jnp.concatenate([W1, W2], axis=0).T
    idx4 = graph_indices.astype(jnp.int32).reshape(S, NW, NCH, CHUNK)
    zeros = jnp.zeros((G, D), jnp.float32)
    parts = []
    for slab in range(S):
        z = _gated_matmul(input, wc, slab)
        parts.append(_segment_sum_sc(z, idx4[slab], zeros))
    return _merge(parts)
